# baseline (speedup 1.0000x reference)
import jax
import jax.numpy as jnp
import numpy as np
from jax.experimental import pallas as pl
from jax.experimental.pallas import tpu as pltpu

_B, _T, _C, _H, _W = 8, 4, 192, 24, 24
_HW = _H * _W
_TC = _T * _C
_K = (_H * _W) // 2
_CHUNK = 256

_GUMBEL = np.asarray(
    jax.random.gumbel(jax.random.key(42), (_B, _HW), dtype=jnp.float32))


def _mae_body(preds_ref, tgts_ref, xrep_ref, gumb_ref, out_ref, d_acc):
    b = pl.program_id(0)
    i = pl.program_id(1)
    n_i = pl.num_programs(1)

    @pl.when(i == 0)
    def _init():
        d_acc[...] = jnp.zeros_like(d_acc)

    p = preds_ref[0]
    t = tgts_ref[0]
    d_acc[...] += jnp.sum(jnp.abs(p - t), axis=0, keepdims=True)

    @pl.when(jnp.logical_and(b == 0, i == n_i - 1))
    def _init_out():
        out_ref[...] = jnp.zeros_like(out_ref)

    @pl.when(i == n_i - 1)
    def _finalize():
        x = xrep_ref[0]
        norm = jnp.sqrt(jnp.sum(x * x, axis=0, keepdims=True))
        scores = gumb_ref[0] - jnp.log(norm + 1e-7)
        col = jnp.transpose(scores)
        j_idx = jax.lax.broadcasted_iota(jnp.int32, (_HW, _HW), 0)
        i_idx = jax.lax.broadcasted_iota(jnp.int32, (_HW, _HW), 1)
        beats = jnp.logical_or(col > scores,
                               jnp.logical_and(col == scores, j_idx < i_idx))
        rank = jnp.sum(beats.astype(jnp.float32), axis=0, keepdims=True)
        masked = jnp.where(rank < float(_K), d_acc[...], 0.0)
        out_ref[...] += jnp.sum(masked, axis=1, keepdims=True)


def kernel(out_preds, out_targets, tl, tv, x_rep, in_x, in_l, in_v, in_n):
    del tl, tv, in_x, in_l, in_v, in_n
    preds = out_preds.reshape(_B, _TC, _HW)
    tgts = out_targets.reshape(_B, _TC, _HW)
    xrep = x_rep.reshape(_B, _C, _HW)
    n_i = _TC // _CHUNK
    total = pl.pallas_call(
        _mae_body,
        grid=(_B, n_i),
        in_specs=[
            pl.BlockSpec((1, _CHUNK, _HW), lambda b, i: (b, i, 0)),
            pl.BlockSpec((1, _CHUNK, _HW), lambda b, i: (b, i, 0)),
            pl.BlockSpec((1, _C, _HW), lambda b, i: (b, 0, 0)),
            pl.BlockSpec((1, 1, _HW), lambda b, i: (b, 0, 0)),
        ],
        out_specs=pl.BlockSpec((1, 1), lambda b, i: (0, 0)),
        out_shape=jax.ShapeDtypeStruct((1, 1), jnp.float32),
        scratch_shapes=[pltpu.VMEM((1, _HW), jnp.float32)],
    )(preds, tgts, xrep, jnp.asarray(_GUMBEL).reshape(_B, 1, _HW))
    return total[0, 0] / np.float32(_B * _T * _K * _C)


# speedup vs baseline: 4.2413x; 4.2413x over previous
import jax
import jax.numpy as jnp
import numpy as np
from jax.experimental import pallas as pl
from jax.experimental.pallas import tpu as pltpu

_B, _T, _C, _H, _W = 8, 4, 192, 24, 24
_HW = _H * _W
_K = _HW // 2


def _threefry2x32(k0, k1, x0, x1):
    rot = [[13, 15, 26, 6], [17, 29, 16, 24]]
    ks = [np.uint32(k0), np.uint32(k1), np.uint32(k0 ^ k1 ^ 0x1BD11BDA)]
    x0 = (x0 + ks[0]).astype(np.uint32)
    x1 = (x1 + ks[1]).astype(np.uint32)

    def rl(v, d):
        return ((v << np.uint32(d)) | (v >> np.uint32(32 - d))).astype(np.uint32)

    for i in range(5):
        for r in rot[i % 2]:
            x0 = (x0 + x1).astype(np.uint32)
            x1 = rl(x1, r) ^ x0
        x0 = (x0 + ks[(i + 1) % 3]).astype(np.uint32)
        x1 = (x1 + ks[(i + 2) % 3] + np.uint32(i + 1)).astype(np.uint32)
    return x0, x1


def _gumbel_const(seed, shape):
    n = int(np.prod(shape))
    x0, x1 = _threefry2x32(np.uint32(seed >> 32), np.uint32(seed & 0xFFFFFFFF),
                           np.zeros(n, np.uint32), np.arange(n, dtype=np.uint32))
    bits = x0 ^ x1
    f = ((bits >> np.uint32(9)) | np.uint32(0x3F800000)).view(np.float32) \
        - np.float32(1.0)
    tiny = np.float32(np.finfo(np.float32).tiny)
    u = np.maximum(tiny, f * (np.float32(1.0) - tiny) + tiny)
    return (-np.log(-np.log(u))).astype(np.float32).reshape(shape)


_GUMBEL = _gumbel_const(42, (_B, _HW))


def _mae_body(preds_ref, tgts_ref, xrep_ref, gumb_ref, out_ref, d_acc):
    b = pl.program_id(0)
    i = pl.program_id(1)
    n_i = pl.num_programs(1)

    @pl.when(i == 0)
    def _init():
        d_acc[...] = jnp.zeros_like(d_acc)

    d_acc[...] += jnp.abs(preds_ref[0] - tgts_ref[0])

    @pl.when(jnp.logical_and(b == 0, i == 0))
    def _init_out():
        out_ref[...] = jnp.zeros_like(out_ref)

    @pl.when(i == n_i - 1)
    def _finalize():
        x = xrep_ref[0]
        norm = jnp.sqrt(jnp.sum(x * x, axis=1, keepdims=True))
        scol = gumb_ref[0] - jnp.log(norm + 1e-7)
        srow = jnp.transpose(scol)
        j_idx = jax.lax.broadcasted_iota(jnp.int32, (_HW, _HW), 0)
        i_idx = jax.lax.broadcasted_iota(jnp.int32, (_HW, _HW), 1)
        beats = jnp.logical_or(scol > srow,
                               jnp.logical_and(scol == srow, j_idx < i_idx))
        rank = jnp.sum(beats.astype(jnp.float32), axis=0, keepdims=True)
        drow = jnp.transpose(jnp.sum(d_acc[...], axis=1, keepdims=True))
        masked = jnp.where(rank < float(_K), drow, 0.0)
        out_ref[...] += jnp.sum(masked, axis=1, keepdims=True)


def kernel(out_preds, out_targets, tl, tv, x_rep, in_x, in_l, in_v, in_n):
    del tl, tv, in_x, in_l, in_v, in_n
    preds = jnp.transpose(out_preds, (0, 1, 3, 4, 2)).reshape(_B, _T * _HW, _C)
    tgts = jnp.transpose(out_targets, (0, 1, 3, 4, 2)).reshape(_B, _T * _HW, _C)
    xrep = jnp.transpose(x_rep, (0, 2, 3, 1)).reshape(_B, _HW, _C)
    gumb = jnp.asarray(_GUMBEL).reshape(_B, _HW, 1)

    total = pl.pallas_call(
        _mae_body,
        grid=(_B, _T),
        in_specs=[
            pl.BlockSpec((1, _HW, _C), lambda b, i: (b, i, 0)),
            pl.BlockSpec((1, _HW, _C), lambda b, i: (b, i, 0)),
            pl.BlockSpec((1, _HW, _C), lambda b, i: (b, 0, 0)),
            pl.BlockSpec((1, _HW, 1), lambda b, i: (b, 0, 0)),
        ],
        out_specs=pl.BlockSpec((1, 1), lambda b, i: (0, 0)),
        out_shape=jax.ShapeDtypeStruct((1, 1), jnp.float32),
        scratch_shapes=[pltpu.VMEM((_HW, _C), jnp.float32)],
    )(preds, tgts, xrep, gumb)
    return total[0, 0] / np.float32(_B * _T * _K * _C)


# speedup vs baseline: 5.6512x; 1.3324x over previous
import jax
import jax.numpy as jnp
import numpy as np
from jax.experimental import pallas as pl
from jax.experimental.pallas import tpu as pltpu

_B, _T, _C, _H, _W = 8, 4, 192, 24, 24
_HW = _H * _W
_K = _HW // 2


def _threefry2x32(k0, k1, x0, x1):
    rot = [[13, 15, 26, 6], [17, 29, 16, 24]]
    ks = [np.uint32(k0), np.uint32(k1), np.uint32(k0 ^ k1 ^ 0x1BD11BDA)]
    x0 = (x0 + ks[0]).astype(np.uint32)
    x1 = (x1 + ks[1]).astype(np.uint32)

    def rl(v, d):
        return ((v << np.uint32(d)) | (v >> np.uint32(32 - d))).astype(np.uint32)

    for i in range(5):
        for r in rot[i % 2]:
            x0 = (x0 + x1).astype(np.uint32)
            x1 = rl(x1, r) ^ x0
        x0 = (x0 + ks[(i + 1) % 3]).astype(np.uint32)
        x1 = (x1 + ks[(i + 2) % 3] + np.uint32(i + 1)).astype(np.uint32)
    return x0, x1


def _gumbel_const(seed, shape):
    n = int(np.prod(shape))
    x0, x1 = _threefry2x32(np.uint32(seed >> 32), np.uint32(seed & 0xFFFFFFFF),
                           np.zeros(n, np.uint32), np.arange(n, dtype=np.uint32))
    bits = x0 ^ x1
    f = ((bits >> np.uint32(9)) | np.uint32(0x3F800000)).view(np.float32) \
        - np.float32(1.0)
    tiny = np.float32(np.finfo(np.float32).tiny)
    u = np.maximum(tiny, f * (np.float32(1.0) - tiny) + tiny)
    return (-np.log(-np.log(u))).astype(np.float32).reshape(shape)


_GUMBEL = _gumbel_const(42, (_B, _HW))


def _mae_body(preds_ref, tgts_ref, xrep_ref, gumb_ref, out_ref, d_acc):
    b = pl.program_id(0)
    i = pl.program_id(1)
    n_i = pl.num_programs(1)

    @pl.when(i == 0)
    def _init():
        d_acc[...] = jnp.zeros_like(d_acc)

    ad = jnp.abs(preds_ref[0] - tgts_ref[0])
    d_acc[...] += ad[:_HW] + ad[_HW:]

    @pl.when(jnp.logical_and(b == 0, i == 0))
    def _init_out():
        out_ref[...] = jnp.zeros_like(out_ref)

    @pl.when(i == n_i - 1)
    def _finalize():
        x = xrep_ref[0]
        norm = jnp.sqrt(jnp.sum(x * x, axis=1, keepdims=True))
        scol = gumb_ref[0] - jnp.log(norm + 1e-7)
        srow = jnp.transpose(scol)
        j_idx = jax.lax.broadcasted_iota(jnp.int32, (_HW, _HW), 0)
        i_idx = jax.lax.broadcasted_iota(jnp.int32, (_HW, _HW), 1)
        beats = jnp.logical_or(scol > srow,
                               jnp.logical_and(scol == srow, j_idx < i_idx))
        rank = jnp.sum(beats.astype(jnp.float32), axis=0, keepdims=True)
        drow = jnp.transpose(jnp.sum(d_acc[...], axis=1, keepdims=True))
        masked = jnp.where(rank < float(_K), drow, 0.0)
        out_ref[...] += jnp.sum(masked, axis=1, keepdims=True)


def kernel(out_preds, out_targets, tl, tv, x_rep, in_x, in_l, in_v, in_n):
    del tl, tv, in_x, in_l, in_v, in_n
    preds = jnp.transpose(out_preds, (0, 1, 3, 4, 2)).reshape(_B, _T * _HW, _C)
    tgts = jnp.transpose(out_targets, (0, 1, 3, 4, 2)).reshape(_B, _T * _HW, _C)
    xrep = jnp.transpose(x_rep, (0, 2, 3, 1)).reshape(_B, _HW, _C)
    gumb = jnp.asarray(_GUMBEL).reshape(_B, _HW, 1)

    total = pl.pallas_call(
        _mae_body,
        grid=(_B, _T // 2),
        in_specs=[
            pl.BlockSpec((1, 2 * _HW, _C), lambda b, i: (b, i, 0)),
            pl.BlockSpec((1, 2 * _HW, _C), lambda b, i: (b, i, 0)),
            pl.BlockSpec((1, _HW, _C), lambda b, i: (b, 0, 0)),
            pl.BlockSpec((1, _HW, 1), lambda b, i: (b, 0, 0)),
        ],
        out_specs=pl.BlockSpec((1, 1), lambda b, i: (0, 0)),
        out_shape=jax.ShapeDtypeStruct((1, 1), jnp.float32),
        scratch_shapes=[pltpu.VMEM((_HW, _C), jnp.float32)],
    )(preds, tgts, xrep, gumb)
    return total[0, 0] / np.float32(_B * _T * _K * _C)
